# baseline (speedup 1.0000x reference)
import numpy as np
import jax
import jax.numpy as jnp
from jax import lax
from jax.experimental import pallas as pl
from jax.experimental.pallas import tpu as pltpu
from jax.experimental.pallas import tpu_sc as plsc

ALPHA = 4
FB = 4
NC, NS = 2, 16


def _copy_body(in_ref, fast_ref):
    fast_ref[...] = in_ref[...]


def _make_sc_gather(C, T, H, W, N, a, b):
    HH = H // 2
    n_chunks = C * N * 2
    n_workers = NC * NS
    per_w = n_chunks // n_workers
    assert n_chunks % n_workers == 0

    mesh = plsc.VectorSubcoreMesh(
        core_axis_name="c", subcore_axis_name="s",
        num_cores=NC, num_subcores=NS,
    )

    def body(frames_hbm, slow_hbm, buf):
        wid = lax.axis_index("s") * NC + lax.axis_index("c")
        for j in range(per_w):
            q = wid * per_w + j
            c = q // (N * 2)
            r = q % (N * 2)
            k = r // 2
            half = r % 2
            t = (k * a) // b
            h0 = half * HH
            pltpu.sync_copy(frames_hbm.at[c, t, pl.ds(h0, HH), :], buf)
            pltpu.sync_copy(buf, slow_hbm.at[c, k, pl.ds(h0, HH), :])

    return pl.kernel(
        body,
        out_type=jax.ShapeDtypeStruct((C, N, H, W), jnp.float32),
        mesh=mesh,
        scratch_types=[pltpu.VMEM((HH, W), jnp.float32)],
    )


def kernel(frames):
    C, T, H, W = frames.shape
    N = T // ALPHA
    a, b = T - 1, N - 1

    idx = np.linspace(0, T - 1, N).astype(np.int64)
    idx_arith = (np.arange(N) * a) // b
    assert np.array_equal(idx, idx_arith), (idx, idx_arith)

    slow = _make_sc_gather(C, T, H, W, N, a, b)(frames)

    fast = pl.pallas_call(
        _copy_body,
        grid=(T // FB,),
        in_specs=[pl.BlockSpec((C, FB, H, W), lambda s: (0, s, 0, 0))],
        out_specs=pl.BlockSpec((C, FB, H, W), lambda s: (0, s, 0, 0)),
        out_shape=jax.ShapeDtypeStruct((C, T, H, W), frames.dtype),
    )(frames)

    return (slow, fast)
